# baseline (speedup 1.0000x reference)
import functools

import jax
import jax.numpy as jnp
from jax import lax
from jax.experimental import pallas as pl
from jax.experimental.pallas import tpu as pltpu
import jax.experimental.pallas.tpu_sc as plsc

D_MODEL = 128
TOTAL_EMB = 150
N_TOKENS = 16384

_NC, _NS, _L = 2, 16, 16
_NW = _NC * _NS
_S_SC = 4096
_TPW = _S_SC // _NW
_BLK = 16
_NBLK = _TPW // _BLK
_KSEG = D_MODEL // _L

_GDN = lax.GatherDimensionNumbers(
    offset_dims=(), collapsed_slice_dims=(0,), start_index_map=(0,))


def _lane_bcast(vec, j):
    jj = jnp.full((_L, 1), j, jnp.int32)
    return lax.gather(vec, jj, _GDN, (1,),
                      mode=lax.GatherScatterMode.PROMISE_IN_BOUNDS)


def _sc_body(cv_hbm, c_hbm, idx_hbm, table_hbm, w_hbm, b_hbm, out_hbm,
             table_v, idx_v, cv_v, c_v, w_v, b_v, obuf, sem):
    wid = lax.axis_index("s") * _NC + lax.axis_index("c")
    base = wid * _TPW

    pltpu.sync_copy(table_hbm, table_v)
    pltpu.sync_copy(idx_hbm.at[pl.ds(base, _TPW)], idx_v)
    pltpu.sync_copy(cv_hbm.at[pl.ds(base, _TPW)], cv_v)
    pltpu.sync_copy(c_hbm.at[pl.ds(base, _TPW)], c_v)
    pltpu.sync_copy(w_hbm, w_v)
    pltpu.sync_copy(b_hbm, b_v)

    wk = [w_v[pl.ds(_L * k, _L)] for k in range(_KSEG)]
    bk = [b_v[pl.ds(_L * k, _L)] for k in range(_KSEG)]
    col = jnp.arange(_L, dtype=jnp.int32)
    cols = [col + _L * k for k in range(_KSEG)]

    @plsc.parallel_loop(0, _NBLK)
    def _blocks(ib):
        t0 = ib * _BLK
        idx16 = idx_v[pl.ds(t0, _BLK)]
        cv16 = cv_v[pl.ds(t0, _BLK)]
        c16 = c_v[pl.ds(t0, _BLK)]
        for j in range(_BLK):
            idx_spl = _lane_bcast(idx16, j)
            cv_spl = _lane_bcast(cv16, j)
            c_spl = _lane_bcast(c16, j)
            m = c_spl > 0.5
            row0 = idx_spl * D_MODEL
            tl = t0 + j
            for k in range(_KSEG):
                g = plsc.load_gather(table_v, [row0 + cols[k]])
                o = jnp.where(m, cv_spl * wk[k] + bk[k], g)
                obuf[pl.ds(tl * D_MODEL + _L * k, _L)] = o

    dst = out_hbm.at[pl.ds(base * D_MODEL, _TPW * D_MODEL)]
    pltpu.async_copy(obuf, dst, sem).wait()


def _sc_part(cv, cm, idx, table_flat, w, b):
    run = pl.kernel(
        _sc_body,
        out_type=jax.ShapeDtypeStruct((_S_SC * D_MODEL,), jnp.float32),
        mesh=plsc.VectorSubcoreMesh(
            core_axis_name="c", subcore_axis_name="s",
            num_cores=_NC, num_subcores=_NS),
        compiler_params=pltpu.CompilerParams(needs_layout_passes=False),
        scratch_types=[
            pltpu.VMEM((TOTAL_EMB * D_MODEL,), jnp.float32),
            pltpu.VMEM((_TPW,), jnp.int32),
            pltpu.VMEM((_TPW,), jnp.float32),
            pltpu.VMEM((_TPW,), jnp.float32),
            pltpu.VMEM((D_MODEL,), jnp.float32),
            pltpu.VMEM((D_MODEL,), jnp.float32),
            pltpu.VMEM((_TPW * D_MODEL,), jnp.float32),
            pltpu.SemaphoreType.DMA,
        ],
    )
    return run(cv, cm, idx, table_flat, w, b).reshape(_S_SC, D_MODEL)


_PAD_EMB = 160
_TBLK = 2048
_NTBLK = (N_TOKENS - _S_SC) // _TBLK


def _tc_body(cv_ref, mask_ref, idx_ref, table_ref, w_ref, b_ref, out_ref):
    cv = cv_ref[0, 0, :]
    mask = mask_ref[0, 0, :]
    idx = idx_ref[0, 0, :]
    cols = jax.lax.broadcasted_iota(jnp.int32, (_TBLK, _PAD_EMB), 1)
    onehot = (cols == idx[:, None]).astype(jnp.float32)
    emb_out = jax.lax.dot_general(
        onehot, table_ref[...],
        dimension_numbers=(((1,), (0,)), ((), ())),
        preferred_element_type=jnp.float32,
    )
    const_out = cv[:, None] * w_ref[0, :][None, :] + b_ref[0, :][None, :]
    out_ref[...] = jnp.where(mask[:, None] > 0.5, const_out, emb_out)


def _tc_part(cv, mask, idx, table_pad, w, b):
    n_tc = N_TOKENS - _S_SC
    cv3 = cv.reshape(_NTBLK, 1, _TBLK)
    m3 = mask.reshape(_NTBLK, 1, _TBLK)
    i3 = idx.reshape(_NTBLK, 1, _TBLK)
    return pl.pallas_call(
        _tc_body,
        grid=(_NTBLK,),
        in_specs=[
            pl.BlockSpec((1, 1, _TBLK), lambda i: (i, 0, 0)),
            pl.BlockSpec((1, 1, _TBLK), lambda i: (i, 0, 0)),
            pl.BlockSpec((1, 1, _TBLK), lambda i: (i, 0, 0)),
            pl.BlockSpec((_PAD_EMB, D_MODEL), lambda i: (0, 0)),
            pl.BlockSpec((1, D_MODEL), lambda i: (0, 0)),
            pl.BlockSpec((1, D_MODEL), lambda i: (0, 0)),
        ],
        out_specs=pl.BlockSpec((_TBLK, D_MODEL), lambda i: (i, 0)),
        out_shape=jax.ShapeDtypeStruct((n_tc, D_MODEL), jnp.float32),
    )(cv3, m3, i3, table_pad, w, b)


@jax.jit
def kernel(const_vals, W_const, b_const, emb_table, is_const, emb_type_idx):
    cv = const_vals.astype(jnp.float32)
    idx = emb_type_idx.astype(jnp.int32)
    w = W_const.reshape(1, D_MODEL).astype(jnp.float32)
    b = b_const.reshape(1, D_MODEL).astype(jnp.float32)
    table = emb_table.astype(jnp.float32)

    sc_out = _sc_part(
        cv[:_S_SC], is_const[:_S_SC].astype(jnp.float32), idx[:_S_SC],
        table.reshape(TOTAL_EMB * D_MODEL),
        w.reshape(D_MODEL), b.reshape(D_MODEL))

    table_pad = jnp.zeros((_PAD_EMB, D_MODEL), jnp.float32).at[:TOTAL_EMB].set(table)
    tc_out = _tc_part(cv[_S_SC:], is_const[_S_SC:].astype(jnp.float32),
                      idx[_S_SC:], table_pad, w, b)

    out = jnp.concatenate([sc_out, tc_out], axis=0)
    return out


# speedup vs baseline: 1.0732x; 1.0732x over previous
import functools

import jax
import jax.numpy as jnp
from jax import lax
from jax.experimental import pallas as pl
from jax.experimental.pallas import tpu as pltpu
import jax.experimental.pallas.tpu_sc as plsc

D_MODEL = 128
TOTAL_EMB = 150
N_TOKENS = 16384

_NC, _NS, _L = 1, 16, 16
_NW = _NC * _NS
_S_SC = 2048
_TPW = _S_SC // _NW
_BLK = 16
_NBLK = _TPW // _BLK
_KSEG = D_MODEL // _L

_GDN = lax.GatherDimensionNumbers(
    offset_dims=(), collapsed_slice_dims=(0,), start_index_map=(0,))


def _lane_bcast(vec, j):
    jj = jnp.full((_L, 1), j, jnp.int32)
    return lax.gather(vec, jj, _GDN, (1,),
                      mode=lax.GatherScatterMode.PROMISE_IN_BOUNDS)


def _sc_body(cv_hbm, c_hbm, idx_hbm, table_hbm, w_hbm, b_hbm, out_hbm,
             table_v, idx_v, cv_v, c_v, w_v, b_v, obuf, sem):
    wid = lax.axis_index("s") * _NC + lax.axis_index("c")
    base = wid * _TPW

    pltpu.sync_copy(table_hbm, table_v)
    pltpu.sync_copy(idx_hbm.at[pl.ds(base, _TPW)], idx_v)
    pltpu.sync_copy(cv_hbm.at[pl.ds(base, _TPW)], cv_v)
    pltpu.sync_copy(c_hbm.at[pl.ds(base, _TPW)], c_v)
    pltpu.sync_copy(w_hbm, w_v)
    pltpu.sync_copy(b_hbm, b_v)

    wk = [w_v[pl.ds(_L * k, _L)] for k in range(_KSEG)]
    bk = [b_v[pl.ds(_L * k, _L)] for k in range(_KSEG)]
    col = jnp.arange(_L, dtype=jnp.int32)
    cols = [col + _L * k for k in range(_KSEG)]

    @plsc.parallel_loop(0, _NBLK)
    def _blocks(ib):
        t0 = ib * _BLK
        idx16 = idx_v[pl.ds(t0, _BLK)]
        cv16 = cv_v[pl.ds(t0, _BLK)]
        c16 = c_v[pl.ds(t0, _BLK)]
        for j in range(_BLK):
            idx_spl = _lane_bcast(idx16, j)
            cv_spl = _lane_bcast(cv16, j)
            c_spl = _lane_bcast(c16, j)
            m = c_spl > 0.5
            row0 = idx_spl * D_MODEL
            tl = t0 + j
            for k in range(_KSEG):
                g = plsc.load_gather(table_v, [row0 + cols[k]])
                o = jnp.where(m, cv_spl * wk[k] + bk[k], g)
                obuf[pl.ds(tl * D_MODEL + _L * k, _L)] = o

    dst = out_hbm.at[pl.ds(base * D_MODEL, _TPW * D_MODEL)]
    pltpu.async_copy(obuf, dst, sem).wait()


def _sc_part(cv, cm, idx, table_flat, w, b):
    run = pl.kernel(
        _sc_body,
        out_type=jax.ShapeDtypeStruct((_S_SC * D_MODEL,), jnp.float32),
        mesh=plsc.VectorSubcoreMesh(
            core_axis_name="c", subcore_axis_name="s",
            num_cores=_NC, num_subcores=_NS),
        compiler_params=pltpu.CompilerParams(needs_layout_passes=False),
        scratch_types=[
            pltpu.VMEM((TOTAL_EMB * D_MODEL,), jnp.float32),
            pltpu.VMEM((_TPW,), jnp.int32),
            pltpu.VMEM((_TPW,), jnp.float32),
            pltpu.VMEM((_TPW,), jnp.float32),
            pltpu.VMEM((D_MODEL,), jnp.float32),
            pltpu.VMEM((D_MODEL,), jnp.float32),
            pltpu.VMEM((_TPW * D_MODEL,), jnp.float32),
            pltpu.SemaphoreType.DMA,
        ],
    )
    return run(cv, cm, idx, table_flat, w, b).reshape(_S_SC, D_MODEL)


_PAD_EMB = 160
_TBLK = 2048
_NTBLK = (N_TOKENS - _S_SC) // _TBLK


def _tc_body(cv_ref, mask_ref, idx_ref, table_ref, w_ref, b_ref, out_ref):
    cv = cv_ref[0, 0, :]
    mask = mask_ref[0, 0, :]
    idx = idx_ref[0, 0, :]
    cols = jax.lax.broadcasted_iota(jnp.int32, (_TBLK, _PAD_EMB), 1)
    onehot = (cols == idx[:, None]).astype(jnp.float32)
    emb_out = jax.lax.dot_general(
        onehot, table_ref[...],
        dimension_numbers=(((1,), (0,)), ((), ())),
        preferred_element_type=jnp.float32,
    )
    const_out = cv[:, None] * w_ref[0, :][None, :] + b_ref[0, :][None, :]
    out_ref[...] = jnp.where(mask[:, None] > 0.5, const_out, emb_out)


def _tc_part(cv, mask, idx, table_pad, w, b):
    n_tc = N_TOKENS - _S_SC
    cv3 = cv.reshape(_NTBLK, 1, _TBLK)
    m3 = mask.reshape(_NTBLK, 1, _TBLK)
    i3 = idx.reshape(_NTBLK, 1, _TBLK)
    return pl.pallas_call(
        _tc_body,
        grid=(_NTBLK,),
        in_specs=[
            pl.BlockSpec((1, 1, _TBLK), lambda i: (i, 0, 0)),
            pl.BlockSpec((1, 1, _TBLK), lambda i: (i, 0, 0)),
            pl.BlockSpec((1, 1, _TBLK), lambda i: (i, 0, 0)),
            pl.BlockSpec((_PAD_EMB, D_MODEL), lambda i: (0, 0)),
            pl.BlockSpec((1, D_MODEL), lambda i: (0, 0)),
            pl.BlockSpec((1, D_MODEL), lambda i: (0, 0)),
        ],
        out_specs=pl.BlockSpec((_TBLK, D_MODEL), lambda i: (i, 0)),
        out_shape=jax.ShapeDtypeStruct((n_tc, D_MODEL), jnp.float32),
    )(cv3, m3, i3, table_pad, w, b)


@jax.jit
def kernel(const_vals, W_const, b_const, emb_table, is_const, emb_type_idx):
    cv = const_vals.astype(jnp.float32)
    idx = emb_type_idx.astype(jnp.int32)
    w = W_const.reshape(1, D_MODEL).astype(jnp.float32)
    b = b_const.reshape(1, D_MODEL).astype(jnp.float32)
    table = emb_table.astype(jnp.float32)

    sc_out = _sc_part(
        cv[:_S_SC], is_const[:_S_SC].astype(jnp.float32), idx[:_S_SC],
        table.reshape(TOTAL_EMB * D_MODEL),
        w.reshape(D_MODEL), b.reshape(D_MODEL))

    table_pad = jnp.zeros((_PAD_EMB, D_MODEL), jnp.float32).at[:TOTAL_EMB].set(table)
    tc_out = _tc_part(cv[_S_SC:], is_const[_S_SC:].astype(jnp.float32),
                      idx[_S_SC:], table_pad, w, b)

    out = jnp.concatenate([sc_out, tc_out], axis=0)
    return out


# speedup vs baseline: 1.4073x; 1.3113x over previous
import functools

import jax
import jax.numpy as jnp
from jax import lax
from jax.experimental import pallas as pl
from jax.experimental.pallas import tpu as pltpu
import jax.experimental.pallas.tpu_sc as plsc

D_MODEL = 128
TOTAL_EMB = 150
N_TOKENS = 16384

_NC, _NS, _L = 1, 16, 16
_NW = _NC * _NS
_S_SC = 2048
_TPW = _S_SC // _NW
_KSEG = D_MODEL // _L


def _sc_body(cv_hbm, c_hbm, idx_hbm, table_hbm, w_hbm, b_hbm, out_hbm,
             table_v, idx_v, cv_v, c_v, w_v, b_v, obuf, sem):
    wid = lax.axis_index("s") * _NC + lax.axis_index("c")
    base = wid * _TPW

    pltpu.sync_copy(table_hbm, table_v)
    pltpu.sync_copy(idx_hbm.at[pl.ds(base, _TPW)], idx_v)
    pltpu.sync_copy(cv_hbm.at[pl.ds(base, _TPW)], cv_v)
    pltpu.sync_copy(c_hbm.at[pl.ds(base, _TPW)], c_v)
    pltpu.sync_copy(w_hbm, w_v)
    pltpu.sync_copy(b_hbm, b_v)

    wk = [w_v[pl.ds(_L * k, _L)] for k in range(_KSEG)]
    bk = [b_v[pl.ds(_L * k, _L)] for k in range(_KSEG)]
    col = jnp.arange(_L, dtype=jnp.int32)
    cols = [col + _L * k for k in range(_KSEG)]

    @plsc.parallel_loop(0, _TPW, unroll=2)
    def _tok(t):
        tvec = jnp.full((_L,), t, jnp.int32)
        idx_spl = plsc.load_gather(idx_v, [tvec])
        cv_spl = plsc.load_gather(cv_v, [tvec])
        c_spl = plsc.load_gather(c_v, [tvec])
        m = c_spl > 0.5
        row0 = idx_spl * D_MODEL
        for k in range(_KSEG):
            g = plsc.load_gather(table_v, [row0 + cols[k]])
            o = jnp.where(m, cv_spl * wk[k] + bk[k], g)
            obuf[pl.ds(t * D_MODEL + _L * k, _L)] = o

    dst = out_hbm.at[pl.ds(base * D_MODEL, _TPW * D_MODEL)]
    pltpu.async_copy(obuf, dst, sem).wait()


def _sc_part(cv, cm, idx, table_flat, w, b):
    run = pl.kernel(
        _sc_body,
        out_type=jax.ShapeDtypeStruct((_S_SC * D_MODEL,), jnp.float32),
        mesh=plsc.VectorSubcoreMesh(
            core_axis_name="c", subcore_axis_name="s",
            num_cores=_NC, num_subcores=_NS),
        compiler_params=pltpu.CompilerParams(needs_layout_passes=False),
        scratch_types=[
            pltpu.VMEM((TOTAL_EMB * D_MODEL,), jnp.float32),
            pltpu.VMEM((_TPW,), jnp.int32),
            pltpu.VMEM((_TPW,), jnp.float32),
            pltpu.VMEM((_TPW,), jnp.float32),
            pltpu.VMEM((D_MODEL,), jnp.float32),
            pltpu.VMEM((D_MODEL,), jnp.float32),
            pltpu.VMEM((_TPW * D_MODEL,), jnp.float32),
            pltpu.SemaphoreType.DMA,
        ],
    )
    return run(cv, cm, idx, table_flat, w, b).reshape(_S_SC, D_MODEL)


_PAD_EMB = 160
_TBLK = 2048
_OFF = _S_SC // _TBLK
_NTBLK = (N_TOKENS - _S_SC) // _TBLK


def _tc_body(cv_ref, mask_ref, idx_ref, table_ref, w_ref, b_ref, out_ref):
    cv = cv_ref[0, 0, :]
    mask = mask_ref[0, 0, :]
    idx = idx_ref[0, 0, :]
    cols = jax.lax.broadcasted_iota(jnp.int32, (_TBLK, _PAD_EMB), 1)
    onehot = (cols == idx[:, None]).astype(jnp.float32)
    emb_out = jax.lax.dot_general(
        onehot, table_ref[...],
        dimension_numbers=(((1,), (0,)), ((), ())),
        preferred_element_type=jnp.float32,
    )
    const_out = cv[:, None] * w_ref[0, :][None, :] + b_ref[0, :][None, :]
    out_ref[...] = jnp.where(mask[:, None] > 0.5, const_out, emb_out)


def _tc_part(cv3, m3, i3, table_pad, w, b):
    return pl.pallas_call(
        _tc_body,
        grid=(_NTBLK,),
        in_specs=[
            pl.BlockSpec((1, 1, _TBLK), lambda i: (i + _OFF, 0, 0)),
            pl.BlockSpec((1, 1, _TBLK), lambda i: (i + _OFF, 0, 0)),
            pl.BlockSpec((1, 1, _TBLK), lambda i: (i + _OFF, 0, 0)),
            pl.BlockSpec((_PAD_EMB, D_MODEL), lambda i: (0, 0)),
            pl.BlockSpec((1, D_MODEL), lambda i: (0, 0)),
            pl.BlockSpec((1, D_MODEL), lambda i: (0, 0)),
        ],
        out_specs=pl.BlockSpec((_TBLK, D_MODEL), lambda i: (i + _OFF, 0)),
        out_shape=jax.ShapeDtypeStruct((N_TOKENS, D_MODEL), jnp.float32),
    )(cv3, m3, i3, table_pad, w, b)


@jax.jit
def kernel(const_vals, W_const, b_const, emb_table, is_const, emb_type_idx):
    cv = const_vals.astype(jnp.float32)
    cm = is_const.astype(jnp.float32)
    idx = emb_type_idx.astype(jnp.int32)
    w = W_const.reshape(1, D_MODEL).astype(jnp.float32)
    b = b_const.reshape(1, D_MODEL).astype(jnp.float32)
    table = emb_table.astype(jnp.float32)

    sc_out = _sc_part(cv, cm, idx, table.reshape(TOTAL_EMB * D_MODEL),
                      w.reshape(D_MODEL), b.reshape(D_MODEL))

    table_pad = jnp.pad(table, ((0, _PAD_EMB - TOTAL_EMB), (0, 0)))
    nblk_all = N_TOKENS // _TBLK
    cv3 = cv.reshape(nblk_all, 1, _TBLK)
    m3 = cm.reshape(nblk_all, 1, _TBLK)
    i3 = idx.reshape(nblk_all, 1, _TBLK)
    tc_out = _tc_part(cv3, m3, i3, table_pad, w, b)

    return lax.dynamic_update_slice(tc_out, sc_out, (0, 0))
